# baseline (speedup 1.0000x reference)
import functools

import jax
import jax.numpy as jnp
from jax import lax
from jax.experimental import pallas as pl
from jax.experimental.pallas import tpu as pltpu
from jax.experimental.pallas import tpu_sc as plsc

B = 16384
K = 20
V = 1000000
S = 5
D = 32
H = 64

NC = 2
NS = 16
NW = NC * NS
ROWS = 2 * B
BPW = B // NW
BLK = 128
NBLK = BPW // BLK
CH = 16
SUBS = BLK // CH
G = CH * K
VP = 4

TBLK = 6400
TGRID = -(-V // TBLK)


QC = TBLK // VP


def _tc_pack_table(emb_t):
  def body(x, out):
    eye = (lax.broadcasted_iota(jnp.int32, (D, D), 0) ==
           lax.broadcasted_iota(jnp.int32, (D, D), 1)).astype(jnp.float32)
    y = lax.dot_general(x[...], eye, (((0,), (0,)), ((), ())),
                        preferred_element_type=jnp.float32)
    out[...] = jnp.concatenate([y[QC * j:QC * (j + 1), :] for j in range(VP)],
                               axis=1)

  return pl.pallas_call(
      body,
      grid=(TGRID,),
      in_specs=[pl.BlockSpec((D, TBLK), lambda i: (0, i))],
      out_specs=pl.BlockSpec((QC, VP * D), lambda i: (i, 0)),
      out_shape=jax.ShapeDtypeStruct((TGRID * QC, VP * D), jnp.float32),
      compiler_params=pltpu.CompilerParams(
          dimension_semantics=("parallel",)),
  )(emb_t)


def _sc_weighted_embed(emb4, idx_a_t, idx_h_t, w_a_t, w_h_t):
  mesh = plsc.VectorSubcoreMesh(core_axis_name="c", subcore_axis_name="s")
  NSEG = (G + 127) // 128

  @functools.partial(
      pl.kernel,
      mesh=mesh,
      out_type=jax.ShapeDtypeStruct((ROWS // VP, VP * D), jnp.float32),
      scratch_types=[
          pltpu.VMEM((K, BLK), jnp.int32),
          pltpu.VMEM((K, BLK), jnp.float32),
          pltpu.VMEM((G,), jnp.int32),
          pltpu.VMEM((G,), jnp.int32),
          pltpu.VMEM((G,), jnp.int32),
          pltpu.VMEM((G,), jnp.int32),
          pltpu.VMEM((G, VP * D), jnp.float32),
          pltpu.VMEM((G, VP * D), jnp.float32),
          pltpu.VMEM((2 * CH // VP, VP * D), jnp.float32),
          pltpu.SemaphoreType.DMA,
          pltpu.SemaphoreType.DMA,
      ],
  )
  def body(emb_hbm, ia_hbm, ih_hbm, wa_hbm, wh_hbm, out_hbm, idx_v, w_v,
           flat_a, flat_b, off_a, off_b, rows_a, rows_b, acc_v, sem_a, sem_b):
    wid = lax.axis_index("s") * NC + lax.axis_index("c")
    base = wid * BPW
    slots = ((flat_a, off_a, rows_a, sem_a), (flat_b, off_b, rows_b, sem_b))

    def build_flat(s, flat, off):
      for k in range(K):
        r = idx_v[k, pl.ds(s * CH, 16)]
        t = ((r >> 6) * 41944) >> 20
        flat[pl.ds(k * CH, 16)] = ((t >> 2) * QC) + r - (t * QC)
        off[pl.ds(k * CH, 16)] = (t & 3) << 5

    def fire(flat, off, rows, sem):
      del off
      cps = []
      o = 0
      for seg in range(NSEG):
        n = min(128, G - seg * 128)
        cps.append(pltpu.async_copy(
            emb_hbm.at[flat.at[pl.ds(o, n)]], rows.at[pl.ds(o, n)], sem))
        o += n
      return cps

    for side, (i_hbm, v_hbm) in enumerate(((ia_hbm, wa_hbm), (ih_hbm, wh_hbm))):

      def block_body(c, carry):
        b0 = base + c * BLK
        pltpu.sync_copy(i_hbm.at[:, pl.ds(b0, BLK)], idx_v)
        pltpu.sync_copy(v_hbm.at[:, pl.ds(b0, BLK)], w_v)

        build_flat(0, slots[0][0], slots[0][1])
        cps = fire(*slots[0])
        for s in range(SUBS):
          flat, off, rows, sem = slots[s % 2]
          if s + 1 < SUBS:
            nxt = slots[(s + 1) % 2]
            build_flat(s + 1, nxt[0], nxt[1])
            nxt_cps = fire(*nxt)
          else:
            nxt_cps = None
          for cp in cps:
            cp.wait()

          r0 = (s % 2) * (CH // VP)
          zero = jnp.zeros((16,), jnp.float32)
          for r in range(CH // VP):
            for h in range(VP * D // 16):
              acc_v[r0 + r, pl.ds(16 * h, 16)] = zero

          def k_step(k, carry2):
            g = k * CH
            kv = w_v[k, pl.ds(s * CH, 16)]
            jv = off[pl.ds(g, 16)]
            for e in range(CH):
              ws = kv[e]
              o = jv[e]
              v0 = rows[g + e, pl.ds(o, 16)] * ws
              v1 = rows[g + e, pl.ds(o + 16, 16)] * ws
              d0 = (e % VP) * D
              plsc.addupdate(acc_v.at[r0 + e // VP, pl.ds(d0, 16)], v0)
              plsc.addupdate(acc_v.at[r0 + e // VP, pl.ds(d0 + 16, 16)], v1)
            return carry2

          lax.fori_loop(0, K, k_step, 0)
          if s % 2 == 1:
            row0 = pl.multiple_of(
                (side * B + b0 + (s - 1) * CH) // VP, 8)
            pltpu.sync_copy(
                acc_v, out_hbm.at[pl.ds(row0, 2 * CH // VP)])
          cps = nxt_cps
        return carry

      lax.fori_loop(0, NBLK, block_body, 0)

  return body(emb4, idx_a_t, idx_h_t, w_a_t, w_h_t)


BS = 2048
GRID = B // BS


def _tc_head(sa_t, wa_t, sh_t, wh_t, e_all, WsT, b_statc, W1aT, W1bT, b1c,
             W2T, b2c, W3T, b3c):
  def body(sa, wa, sh, wh, ea, eh, wst, bst, w1a, w1b, bb1, w2, bb2, w3,
           bb3, out):
    f32 = jnp.float32
    dot = lambda x, y: lax.dot_general(x, y, (((1,), (0,)), ((), ())),
                                       preferred_element_type=f32)

    def pooled(st, wt, et):
      w = wt[...]
      ws = jnp.concatenate(
          [jnp.sum(st[pl.ds(s * K, K), :] * w, axis=0, keepdims=True)
           for s in range(S)], axis=0)
      p = dot(wst[...], ws) + et[...].T
      return p + bst[...] * jnp.sum(w, axis=0, keepdims=True)

    pa = pooled(sa, wa, ea)
    ph = pooled(sh, wh, eh)
    h1 = jnp.maximum(dot(w1a[...], pa) + dot(w1b[...], ph) + bb1[...], 0.0)
    h2 = jnp.maximum(dot(w2[...], h1) + bb2[...], 0.0)
    out[...] = dot(w3[...], h2) + bb3[...]

  KS = K * S
  in_specs = [
      pl.BlockSpec((KS, BS), lambda i: (0, i)),
      pl.BlockSpec((K, BS), lambda i: (0, i)),
      pl.BlockSpec((KS, BS), lambda i: (0, i)),
      pl.BlockSpec((K, BS), lambda i: (0, i)),
      pl.BlockSpec((BS, D), lambda i: (i, 0)),
      pl.BlockSpec((BS, D), lambda i: (i + GRID, 0)),
      pl.BlockSpec((D, S), lambda i: (0, 0)),
      pl.BlockSpec((D, 1), lambda i: (0, 0)),
      pl.BlockSpec((H, D), lambda i: (0, 0)),
      pl.BlockSpec((H, D), lambda i: (0, 0)),
      pl.BlockSpec((H, 1), lambda i: (0, 0)),
      pl.BlockSpec((H, H), lambda i: (0, 0)),
      pl.BlockSpec((H, 1), lambda i: (0, 0)),
      pl.BlockSpec((1, H), lambda i: (0, 0)),
      pl.BlockSpec((1, 1), lambda i: (0, 0)),
  ]
  return pl.pallas_call(
      body,
      grid=(GRID,),
      in_specs=in_specs,
      out_specs=pl.BlockSpec((1, BS), lambda i: (0, i)),
      out_shape=jax.ShapeDtypeStruct((1, B), jnp.float32),
  )(sa_t, wa_t, sh_t, wh_t, e_all, e_all, WsT, b_statc, W1aT, W1bT, b1c,
    W2T, b2c, W3T, b3c)


def kernel(away_indices, home_indices, away_stats, home_stats, away_weights,
           home_weights, W_stat, b_stat, emb, W1, b1, W2, b2, W3, b3):
  emb4 = _tc_pack_table(emb.T)
  e4 = _sc_weighted_embed(emb4,
                          away_indices.T.astype(jnp.int32),
                          home_indices.T.astype(jnp.int32),
                          away_weights.T, home_weights.T)
  e_all = e4.reshape(ROWS, D)

  sa_t = away_stats.transpose(2, 1, 0).reshape(S * K, B)
  sh_t = home_stats.transpose(2, 1, 0).reshape(S * K, B)
  out = _tc_head(sa_t, away_weights.T, sh_t, home_weights.T, e_all,
                 W_stat.T, b_stat.reshape(D, 1), W1[:D].T, W1[D:].T,
                 b1.reshape(H, 1), W2.T, b2.reshape(H, 1), W3.T,
                 b3.reshape(1, 1))
  return out[0]


# speedup vs baseline: 1.0152x; 1.0152x over previous
import functools

import jax
import jax.numpy as jnp
from jax import lax
from jax.experimental import pallas as pl
from jax.experimental.pallas import tpu as pltpu
from jax.experimental.pallas import tpu_sc as plsc

B = 16384
K = 20
V = 1000000
S = 5
D = 32
H = 64

NC = 2
NS = 16
NW = NC * NS
ROWS = 2 * B
BPW = B // NW
BLK = 128
NBLK = BPW // BLK
CH = 16
SUBS = BLK // CH
G = CH * K
VP = 4

TBLK = 12800
TGRID = -(-V // TBLK)


QC = TBLK // VP


def _tc_pack_table(emb_t):
  def body(x, out):
    eye = (lax.broadcasted_iota(jnp.int32, (D, D), 0) ==
           lax.broadcasted_iota(jnp.int32, (D, D), 1)).astype(jnp.float32)
    y = lax.dot_general(x[...], eye, (((0,), (0,)), ((), ())),
                        preferred_element_type=jnp.float32)
    out[...] = jnp.concatenate([y[QC * j:QC * (j + 1), :] for j in range(VP)],
                               axis=1)

  return pl.pallas_call(
      body,
      grid=(TGRID,),
      in_specs=[pl.BlockSpec((D, TBLK), lambda i: (0, i))],
      out_specs=pl.BlockSpec((QC, VP * D), lambda i: (i, 0)),
      out_shape=jax.ShapeDtypeStruct((TGRID * QC, VP * D), jnp.float32),
      compiler_params=pltpu.CompilerParams(
          dimension_semantics=("parallel",)),
  )(emb_t)


def _sc_weighted_embed(emb4, idx_a_t, idx_h_t, w_a_t, w_h_t):
  mesh = plsc.VectorSubcoreMesh(core_axis_name="c", subcore_axis_name="s")
  NSEG = (G + 127) // 128

  @functools.partial(
      pl.kernel,
      mesh=mesh,
      out_type=jax.ShapeDtypeStruct((ROWS // VP, VP * D), jnp.float32),
      scratch_types=[
          pltpu.VMEM((K, BLK), jnp.int32),
          pltpu.VMEM((K, BLK), jnp.float32),
          pltpu.VMEM((G,), jnp.int32),
          pltpu.VMEM((G,), jnp.int32),
          pltpu.VMEM((G,), jnp.int32),
          pltpu.VMEM((G,), jnp.int32),
          pltpu.VMEM((G, VP * D), jnp.float32),
          pltpu.VMEM((G, VP * D), jnp.float32),
          pltpu.VMEM((2 * CH // VP, VP * D), jnp.float32),
          pltpu.SemaphoreType.DMA,
          pltpu.SemaphoreType.DMA,
      ],
  )
  def body(emb_hbm, ia_hbm, ih_hbm, wa_hbm, wh_hbm, out_hbm, idx_v, w_v,
           flat_a, flat_b, off_a, off_b, rows_a, rows_b, acc_v, sem_a, sem_b):
    wid = lax.axis_index("s") * NC + lax.axis_index("c")
    base = wid * BPW
    slots = ((flat_a, off_a, rows_a, sem_a), (flat_b, off_b, rows_b, sem_b))

    def build_flat(s, flat, off):
      for k in range(K):
        r = idx_v[k, pl.ds(s * CH, 16)]
        t = ((r >> 7) * 41944) >> 20
        flat[pl.ds(k * CH, 16)] = ((t >> 2) * QC) + r - (t * QC)
        off[pl.ds(k * CH, 16)] = (t & 3) << 5

    def fire(flat, off, rows, sem):
      del off
      cps = []
      o = 0
      for seg in range(NSEG):
        n = min(128, G - seg * 128)
        cps.append(pltpu.async_copy(
            emb_hbm.at[flat.at[pl.ds(o, n)]], rows.at[pl.ds(o, n)], sem))
        o += n
      return cps

    for side, (i_hbm, v_hbm) in enumerate(((ia_hbm, wa_hbm), (ih_hbm, wh_hbm))):

      def block_body(c, carry):
        b0 = base + c * BLK
        pltpu.sync_copy(i_hbm.at[:, pl.ds(b0, BLK)], idx_v)
        pltpu.sync_copy(v_hbm.at[:, pl.ds(b0, BLK)], w_v)

        build_flat(0, slots[0][0], slots[0][1])
        cps = fire(*slots[0])
        for s in range(SUBS):
          flat, off, rows, sem = slots[s % 2]
          if s + 1 < SUBS:
            nxt = slots[(s + 1) % 2]
            build_flat(s + 1, nxt[0], nxt[1])
            nxt_cps = fire(*nxt)
          else:
            nxt_cps = None
          for cp in cps:
            cp.wait()

          r0 = (s % 2) * (CH // VP)
          zero = jnp.zeros((16,), jnp.float32)
          for r in range(CH // VP):
            for h in range(VP * D // 16):
              acc_v[r0 + r, pl.ds(16 * h, 16)] = zero

          def k_step(k, carry2):
            g = k * CH
            kv = w_v[k, pl.ds(s * CH, 16)]
            jv = off[pl.ds(g, 16)]
            for e in range(CH):
              ws = kv[e]
              o = jv[e]
              v0 = rows[g + e, pl.ds(o, 16)] * ws
              v1 = rows[g + e, pl.ds(o + 16, 16)] * ws
              d0 = (e % VP) * D
              plsc.addupdate(acc_v.at[r0 + e // VP, pl.ds(d0, 16)], v0)
              plsc.addupdate(acc_v.at[r0 + e // VP, pl.ds(d0 + 16, 16)], v1)
            return carry2

          lax.fori_loop(0, K, k_step, 0)
          if s % 2 == 1:
            row0 = pl.multiple_of(
                (side * B + b0 + (s - 1) * CH) // VP, 8)
            pltpu.sync_copy(
                acc_v, out_hbm.at[pl.ds(row0, 2 * CH // VP)])
          cps = nxt_cps
        return carry

      lax.fori_loop(0, NBLK, block_body, 0)

  return body(emb4, idx_a_t, idx_h_t, w_a_t, w_h_t)


BS = 2048
GRID = B // BS


def _tc_head(sa_t, wa_t, sh_t, wh_t, e_all, WsT, b_statc, W1aT, W1bT, b1c,
             W2T, b2c, W3T, b3c):
  def body(sa, wa, sh, wh, ea, eh, wst, bst, w1a, w1b, bb1, w2, bb2, w3,
           bb3, out):
    f32 = jnp.float32
    dot = lambda x, y: lax.dot_general(x, y, (((1,), (0,)), ((), ())),
                                       preferred_element_type=f32)

    def pooled(st, wt, et):
      w = wt[...]
      ws = jnp.concatenate(
          [jnp.sum(st[pl.ds(s * K, K), :] * w, axis=0, keepdims=True)
           for s in range(S)], axis=0)
      p = dot(wst[...], ws) + et[...].T
      return p + bst[...] * jnp.sum(w, axis=0, keepdims=True)

    pa = pooled(sa, wa, ea)
    ph = pooled(sh, wh, eh)
    h1 = jnp.maximum(dot(w1a[...], pa) + dot(w1b[...], ph) + bb1[...], 0.0)
    h2 = jnp.maximum(dot(w2[...], h1) + bb2[...], 0.0)
    out[...] = dot(w3[...], h2) + bb3[...]

  KS = K * S
  in_specs = [
      pl.BlockSpec((KS, BS), lambda i: (0, i)),
      pl.BlockSpec((K, BS), lambda i: (0, i)),
      pl.BlockSpec((KS, BS), lambda i: (0, i)),
      pl.BlockSpec((K, BS), lambda i: (0, i)),
      pl.BlockSpec((BS, D), lambda i: (i, 0)),
      pl.BlockSpec((BS, D), lambda i: (i + GRID, 0)),
      pl.BlockSpec((D, S), lambda i: (0, 0)),
      pl.BlockSpec((D, 1), lambda i: (0, 0)),
      pl.BlockSpec((H, D), lambda i: (0, 0)),
      pl.BlockSpec((H, D), lambda i: (0, 0)),
      pl.BlockSpec((H, 1), lambda i: (0, 0)),
      pl.BlockSpec((H, H), lambda i: (0, 0)),
      pl.BlockSpec((H, 1), lambda i: (0, 0)),
      pl.BlockSpec((1, H), lambda i: (0, 0)),
      pl.BlockSpec((1, 1), lambda i: (0, 0)),
  ]
  return pl.pallas_call(
      body,
      grid=(GRID,),
      in_specs=in_specs,
      out_specs=pl.BlockSpec((1, BS), lambda i: (0, i)),
      out_shape=jax.ShapeDtypeStruct((1, B), jnp.float32),
  )(sa_t, wa_t, sh_t, wh_t, e_all, e_all, WsT, b_statc, W1aT, W1bT, b1c,
    W2T, b2c, W3T, b3c)


def kernel(away_indices, home_indices, away_stats, home_stats, away_weights,
           home_weights, W_stat, b_stat, emb, W1, b1, W2, b2, W3, b3):
  emb4 = _tc_pack_table(emb.T)
  e4 = _sc_weighted_embed(emb4,
                          away_indices.T.astype(jnp.int32),
                          home_indices.T.astype(jnp.int32),
                          away_weights.T, home_weights.T)
  e_all = e4.reshape(ROWS, D)

  sa_t = away_stats.transpose(2, 1, 0).reshape(S * K, B)
  sh_t = home_stats.transpose(2, 1, 0).reshape(S * K, B)
  out = _tc_head(sa_t, away_weights.T, sh_t, home_weights.T, e_all,
                 W_stat.T, b_stat.reshape(D, 1), W1[:D].T, W1[D:].T,
                 b1.reshape(H, 1), W2.T, b2.reshape(H, 1), W3.T,
                 b3.reshape(1, 1))
  return out[0]
